# baseline (speedup 1.0000x reference)
import jax
import jax.numpy as jnp
from jax import lax
from jax.experimental import pallas as pl
from jax.experimental.pallas import tpu as pltpu
from jax.experimental.pallas import tpu_sc as plsc

D = 64
C = 128
NC = 2
NS = 16
NW = NC * NS
NBUF = 4
HIST = 50
BATCH = 16384
ITEMS = HIST * (BATCH // C)
IPW = ITEMS // NW
NG = IPW // NBUF


def _emb_body(idxt_hbm, tab_hbm, out_hbm, idx_v, g_v, t_v, gsems, osems):
    wid = lax.axis_index("s") * NC + lax.axis_index("c")
    base = wid * IPW
    pltpu.sync_copy(idxt_hbm.at[wid], idx_v)

    def gather(jl, b):
        return pltpu.make_async_copy(
            tab_hbm.at[idx_v.at[jl]], g_v.at[b], gsems.at[b])

    def outcopy(jl, b):
        t = base + jl
        h = t // C
        tb = t % C
        return pltpu.make_async_copy(
            t_v.at[b], out_hbm.at[h, pl.ds(0, 8), tb], osems.at[b])

    def transpose(b):
        def dbody(d, carry):
            dvec = jnp.zeros((16,), jnp.int32) + d
            for c in range(8):
                rvec = lax.iota(jnp.int32, 16) + (16 * c)
                vec = plsc.load_gather(g_v.at[b], [rvec, dvec])
                t_v[b, d // 8, d % 8, pl.ds(c * 16, 16)] = vec
            return carry

        lax.fori_loop(0, D, dbody, 0)

    for b in range(NBUF):
        gather(b, b).start()

    def group(g, carry):
        for b in range(NBUF):
            jl = g * NBUF + b
            gather(jl, b).wait()

            @pl.when(g > 0)
            def _():
                outcopy(jl - NBUF, b).wait()

            transpose(b)

            @pl.when(g < NG - 1)
            def _():
                gather(jl + NBUF, b).start()

            outcopy(jl, b).start()
        return carry

    lax.fori_loop(0, NG, group, 0)

    last = (NG - 1) * NBUF
    for b in range(NBUF):
        outcopy(last + b, b).wait()


def kernel(input_, weight):
    bsz, hist = input_.shape
    nb = bsz // C
    idxt = input_.T.astype(jnp.int32).reshape(NW, IPW, C)
    mesh = plsc.VectorSubcoreMesh(core_axis_name="c", subcore_axis_name="s")
    k = pl.kernel(
        _emb_body,
        mesh=mesh,
        out_type=jax.ShapeDtypeStruct((hist, 8, nb, 8, C), jnp.float32),
        scratch_types=[
            pltpu.VMEM((IPW, C), jnp.int32),
            pltpu.VMEM((NBUF, C, D), jnp.float32),
            pltpu.VMEM((NBUF, 8, 8, C), jnp.float32),
            pltpu.SemaphoreType.DMA((NBUF,)),
            pltpu.SemaphoreType.DMA((NBUF,)),
        ],
        compiler_params=pltpu.CompilerParams(
            use_tc_tiling_on_sc=False, needs_layout_passes=False),
    )
    out = k(idxt, weight)
    return out.transpose(2, 4, 0, 1, 3).reshape(bsz, hist, D)


# speedup vs baseline: 1.8047x; 1.8047x over previous
import jax
import jax.numpy as jnp
from jax import lax
from jax.experimental import pallas as pl
from jax.experimental.pallas import tpu as pltpu
from jax.experimental.pallas import tpu_sc as plsc

D = 64
C = 128
NC = 2
NS = 16
NW = NC * NS
NBUF = 4
HIST = 50
BATCH = 16384
ITEMS = HIST * (BATCH // C)
IPW = ITEMS // NW
NG = IPW // NBUF
TP = C + 1


def _emb_body(idxt_hbm, tab_hbm, out_hbm, idx_v, g_v, t_v, gsems, osems):
    wid = lax.axis_index("s") * NC + lax.axis_index("c")
    base = wid * IPW
    pltpu.sync_copy(idxt_hbm.at[wid], idx_v)

    def gather(jl, b):
        return pltpu.make_async_copy(
            tab_hbm.at[idx_v.at[jl]], g_v.at[b], gsems.at[b])

    def outcopy(jl, b):
        t = base + jl
        h = t // C
        tb = t % C
        return pltpu.make_async_copy(
            t_v.at[b, pl.ds(0, 8), pl.ds(0, 8), pl.ds(0, C)],
            out_hbm.at[h, pl.ds(0, 8), tb], osems.at[b])

    iota16 = lax.iota(jnp.int32, 16)

    def transpose(b):
        t3 = t_v.at[b]

        def sbody(sb, carry):
            sbvec = jnp.zeros((16,), jnp.int32) + sb
            for c in range(4):
                vec = g_v[b, sb, pl.ds(c * 16, 16)]
                plsc.store_scatter(
                    t3, [(iota16 + 16 * c) // 8, (iota16 + 16 * c) % 8,
                         sbvec], vec)
            return carry

        lax.fori_loop(0, C, sbody, 0)

    for b in range(NBUF):
        gather(b, b).start()

    def group(g, carry):
        for b in range(NBUF):
            jl = g * NBUF + b
            gather(jl, b).wait()

            @pl.when(g > 0)
            def _():
                outcopy(jl - NBUF, b).wait()

            transpose(b)

            @pl.when(g < NG - 1)
            def _():
                gather(jl + NBUF, b).start()

            outcopy(jl, b).start()
        return carry

    lax.fori_loop(0, NG, group, 0)

    last = (NG - 1) * NBUF
    for b in range(NBUF):
        outcopy(last + b, b).wait()


def kernel(input_, weight):
    bsz, hist = input_.shape
    nb = bsz // C
    idxt = input_.T.astype(jnp.int32).reshape(NW, IPW, C)
    mesh = plsc.VectorSubcoreMesh(core_axis_name="c", subcore_axis_name="s")
    k = pl.kernel(
        _emb_body,
        mesh=mesh,
        out_type=jax.ShapeDtypeStruct((hist, 8, nb, 8, C), jnp.float32),
        scratch_types=[
            pltpu.VMEM((IPW, C), jnp.int32),
            pltpu.VMEM((NBUF, C, D), jnp.float32),
            pltpu.VMEM((NBUF, 8, 8, TP), jnp.float32),
            pltpu.SemaphoreType.DMA((NBUF,)),
            pltpu.SemaphoreType.DMA((NBUF,)),
        ],
        compiler_params=pltpu.CompilerParams(
            use_tc_tiling_on_sc=False, needs_layout_passes=False),
    )
    out = k(idxt, weight)
    return out.transpose(2, 4, 0, 1, 3).reshape(bsz, hist, D)


# speedup vs baseline: 1.8067x; 1.0011x over previous
import jax
import jax.numpy as jnp
from jax import lax
from jax.experimental import pallas as pl
from jax.experimental.pallas import tpu as pltpu
from jax.experimental.pallas import tpu_sc as plsc

D = 64
C = 128
NC = 2
NS = 16
NW = NC * NS
NBUF = 4
HIST = 50
BATCH = 16384
ITEMS = HIST * (BATCH // C)
IPW = ITEMS // NW
NG = IPW // NBUF
TP = C + 1


def _emb_body(idxt_hbm, tab_hbm, out_hbm, idx_v, g_v, t_v, gsems, osems):
    wid = lax.axis_index("s") * NC + lax.axis_index("c")
    base = wid * IPW
    pltpu.sync_copy(idxt_hbm.at[wid], idx_v)

    def gather(jl, b):
        return pltpu.make_async_copy(
            tab_hbm.at[idx_v.at[jl]], g_v.at[b], gsems.at[b])

    def outcopy(jl, b):
        t = base + jl
        h = t // C
        tb = t % C
        return pltpu.make_async_copy(
            t_v.at[b, pl.ds(0, 8), pl.ds(0, 8), pl.ds(0, C)],
            out_hbm.at[h, pl.ds(0, 8), tb], osems.at[b])

    iota16 = lax.iota(jnp.int32, 16)

    def transpose(b):
        t3 = t_v.at[b]

        def sblk(s, carry):
            sb0 = s * 16
            for k in range(16):
                sb = sb0 + k
                sbvec = jnp.zeros((16,), jnp.int32) + sb
                for c in range(4):
                    vec = g_v[b, sb, pl.ds(c * 16, 16)]
                    plsc.store_scatter(
                        t3, [(iota16 + 16 * c) // 8, (iota16 + 16 * c) % 8,
                             sbvec], vec)
            return carry

        lax.fori_loop(0, C // 16, sblk, 0)

    for b in range(NBUF):
        gather(b, b).start()

    def group(g, carry):
        for b in range(NBUF):
            jl = g * NBUF + b
            gather(jl, b).wait()

            @pl.when(g > 0)
            def _():
                outcopy(jl - NBUF, b).wait()

            transpose(b)

            @pl.when(jl + NBUF < IPW)
            def _():
                gather(jl + NBUF, b).start()

            outcopy(jl, b).start()
        return carry

    lax.fori_loop(0, NG, group, 0)

    last = (NG - 1) * NBUF
    for b in range(NBUF):
        outcopy(last + b, b).wait()


def kernel(input_, weight):
    bsz, hist = input_.shape
    nb = bsz // C
    idxt = input_.T.astype(jnp.int32).reshape(NW, IPW, C)
    mesh = plsc.VectorSubcoreMesh(core_axis_name="c", subcore_axis_name="s")
    k = pl.kernel(
        _emb_body,
        mesh=mesh,
        out_type=jax.ShapeDtypeStruct((hist, 8, nb, 8, C), jnp.float32),
        scratch_types=[
            pltpu.VMEM((IPW, C), jnp.int32),
            pltpu.VMEM((NBUF, C, D), jnp.float32),
            pltpu.VMEM((NBUF, 8, 8, TP), jnp.float32),
            pltpu.SemaphoreType.DMA((NBUF,)),
            pltpu.SemaphoreType.DMA((NBUF,)),
        ],
        compiler_params=pltpu.CompilerParams(
            use_tc_tiling_on_sc=False, needs_layout_passes=False),
    )
    out = k(idxt, weight)
    return out.transpose(2, 4, 0, 1, 3).reshape(bsz, hist, D)


# speedup vs baseline: 2.4263x; 1.3429x over previous
import jax
import jax.numpy as jnp
from jax import lax
from jax.experimental import pallas as pl
from jax.experimental.pallas import tpu as pltpu
from jax.experimental.pallas import tpu_sc as plsc

D = 64
C = 128
NC = 2
NS = 16
NW = NC * NS
NBUF = 4
HIST = 50
BATCH = 16384
ITEMS = HIST * (BATCH // C)
IPW = ITEMS // NW
NG = IPW // NBUF
TP = C + 1


def _emb_body(idxt_hbm, tab_hbm, out_hbm, idx_v, g_v, t_v, gsems, osems):
    wid = lax.axis_index("s") * NC + lax.axis_index("c")
    base = wid * IPW
    pltpu.sync_copy(idxt_hbm.at[wid], idx_v)

    def gather(jl, b):
        return pltpu.make_async_copy(
            tab_hbm.at[idx_v.at[jl]], g_v.at[b], gsems.at[b])

    def outcopy(jl, b):
        t = base + jl
        h = t // C
        tb = t % C
        return pltpu.make_async_copy(
            t_v.at[b, pl.ds(0, 8), pl.ds(0, 8), pl.ds(0, C)],
            out_hbm.at[h, pl.ds(0, 8), tb], osems.at[b])

    iota16 = lax.iota(jnp.int32, 16)

    def transpose(b):
        t3 = t_v.at[b]

        @plsc.parallel_loop(0, C, step=1, unroll=4)
        def _(sb):
            sbvec = jnp.zeros((16,), jnp.int32) + sb
            for c in range(4):
                vec = g_v[b, sb, pl.ds(c * 16, 16)]
                plsc.store_scatter(
                    t3, [(iota16 + 16 * c) // 8, (iota16 + 16 * c) % 8,
                         sbvec], vec)

    for b in range(NBUF):
        gather(b, b).start()

    def group(g, carry):
        for b in range(NBUF):
            jl = g * NBUF + b
            gather(jl, b).wait()

            @pl.when(g > 0)
            def _():
                outcopy(jl - NBUF, b).wait()

            transpose(b)

            @pl.when(jl + NBUF < IPW)
            def _():
                gather(jl + NBUF, b).start()

            outcopy(jl, b).start()
        return carry

    lax.fori_loop(0, NG, group, 0)

    last = (NG - 1) * NBUF
    for b in range(NBUF):
        outcopy(last + b, b).wait()


def kernel(input_, weight):
    bsz, hist = input_.shape
    nb = bsz // C
    idxt = input_.T.astype(jnp.int32).reshape(NW, IPW, C)
    mesh = plsc.VectorSubcoreMesh(core_axis_name="c", subcore_axis_name="s")
    k = pl.kernel(
        _emb_body,
        mesh=mesh,
        out_type=jax.ShapeDtypeStruct((hist, 8, nb, 8, C), jnp.float32),
        scratch_types=[
            pltpu.VMEM((IPW, C), jnp.int32),
            pltpu.VMEM((NBUF, C, D), jnp.float32),
            pltpu.VMEM((NBUF, 8, 8, TP), jnp.float32),
            pltpu.SemaphoreType.DMA((NBUF,)),
            pltpu.SemaphoreType.DMA((NBUF,)),
        ],
        compiler_params=pltpu.CompilerParams(
            use_tc_tiling_on_sc=False, needs_layout_passes=False),
    )
    out = k(idxt, weight)
    return out.transpose(2, 4, 0, 1, 3).reshape(bsz, hist, D)
